# baseline (device time: 28071 ns/iter reference)
import jax
import jax.numpy as jnp
from jax import lax
from jax.experimental import pallas as pl
from jax.experimental.pallas import tpu as pltpu

N_DEV = 16
N_CHUNKS = 4


def kernel(x, router_W, route_idx, expert_W, shared_W):
    n_tokens, d_model = x.shape
    n_local_e, _, d_hidden = expert_W.shape
    m_per = n_tokens // N_DEV
    rows_per_chunk = n_tokens // N_CHUNKS
    blocks_per_chunk = N_DEV // N_CHUNKS

    def body(x_ref, rw_ref, idx_ref, ew_ref, sw_ref, out_ref,
             coeff_ref, xw_ref, sendbuf_ref, gather_ref,
             send_sems, recv_sems):
        my = lax.axis_index("i")

        scores = jnp.dot(x_ref[:], rw_ref[:],
                         preferred_element_type=jnp.float32)
        s_max = jnp.max(scores, axis=1, keepdims=True)
        p = jnp.exp(scores - s_max)
        probs = p / jnp.sum(p, axis=1, keepdims=True)
        route = idx_ref[:]
        iota_e = lax.broadcasted_iota(jnp.int32, scores.shape, 1)
        prob_sel = jnp.sum(jnp.where(route == iota_e, probs, 0.0),
                           axis=1, keepdims=True)

        coeff_ref[:] = jnp.concatenate(
            [jnp.where(route == my * n_local_e + le, prob_sel, 0.0)
             for le in range(n_local_e)], axis=1)

        for j in range(N_DEV):
            dst = (my + 1 + j) % N_DEV
            xblk = x_ref[pl.ds(dst * m_per, m_per), :]
            cblk = coeff_ref[pl.ds(dst * m_per, m_per), :]
            xw_ref[j * m_per:(j + 1) * m_per, :] = jnp.concatenate(
                [xblk * cblk[:, le:le + 1] for le in range(n_local_e)],
                axis=1)

        descs = []
        for c in range(N_CHUNKS):
            r0 = c * rows_per_chunk
            pblk = jnp.zeros((rows_per_chunk, d_hidden), jnp.float32)
            for le in range(n_local_e):
                pblk = pblk + jnp.dot(
                    xw_ref[r0:r0 + rows_per_chunk,
                           le * d_model:(le + 1) * d_model],
                    ew_ref[le],
                    preferred_element_type=jnp.float32)
            sendbuf_ref[r0:r0 + rows_per_chunk, :] = pblk
            for j in range(c * blocks_per_chunk, (c + 1) * blocks_per_chunk):
                k = j + 1
                if k < N_DEV:
                    desc = pltpu.make_async_remote_copy(
                        src_ref=sendbuf_ref.at[pl.ds(j * m_per, m_per)],
                        dst_ref=gather_ref.at[k],
                        send_sem=send_sems.at[k],
                        recv_sem=recv_sems.at[k],
                        device_id=((my + k) % N_DEV,),
                        device_id_type=pl.DeviceIdType.MESH,
                    )
                    desc.start()
                    descs.append(desc)
                else:
                    gather_ref[0] = sendbuf_ref[j * m_per:(j + 1) * m_per, :]

        shared_blk = jnp.dot(x_ref[pl.ds(my * m_per, m_per), :], sw_ref[:],
                             preferred_element_type=jnp.float32)

        for desc in descs:
            desc.wait_recv()
        for desc in descs:
            desc.wait_send()

        out_ref[:] = shared_blk + jnp.sum(gather_ref[:], axis=0)

    return pl.pallas_call(
        body,
        out_shape=jax.ShapeDtypeStruct((m_per, d_hidden), jnp.float32),
        in_specs=[pl.BlockSpec(memory_space=pltpu.VMEM)] * 5,
        out_specs=pl.BlockSpec(memory_space=pltpu.VMEM),
        scratch_shapes=[
            pltpu.VMEM((n_tokens, n_local_e), jnp.float32),
            pltpu.VMEM((n_tokens, n_local_e * d_model), jnp.float32),
            pltpu.VMEM((n_tokens, d_hidden), jnp.float32),
            pltpu.VMEM((N_DEV, m_per, d_hidden), jnp.float32),
            pltpu.SemaphoreType.DMA((N_DEV,)),
            pltpu.SemaphoreType.DMA((N_DEV,)),
        ],
    )(x, router_W, route_idx, expert_W, shared_W)


# device time: 18768 ns/iter; 1.4957x vs baseline; 1.4957x over previous
import jax
import jax.numpy as jnp
from jax import lax
from jax.experimental import pallas as pl
from jax.experimental.pallas import tpu as pltpu

N_DEV = 16
N_CHUNKS = 4


def kernel(x, router_W, route_idx, expert_W, shared_W):
    n_tokens, d_model = x.shape
    n_local_e, _, d_hidden = expert_W.shape
    m_per = n_tokens // N_DEV
    rows_per_chunk = n_tokens // N_CHUNKS
    blocks_per_chunk = N_DEV // N_CHUNKS

    def body(x_ref, rw_ref, idx_ref, ew_ref, sw_ref, out_ref,
             coeff_ref, xw_ref, ew_bf_ref, sendbuf_ref, gather_ref,
             send_sems, recv_sems, ready_sems):
        my = lax.axis_index("i")

        for k in range(1, N_DEV):
            pl.semaphore_signal(
                ready_sems.at[N_DEV - k], inc=1,
                device_id=((my + k) % N_DEV,),
                device_id_type=pl.DeviceIdType.MESH,
            )
        barrier_sem = pltpu.get_barrier_semaphore()
        pl.semaphore_signal(barrier_sem, 1)
        pl.semaphore_wait(barrier_sem, 1)

        for le in range(n_local_e):
            ew_bf_ref[le] = ew_ref[le].astype(jnp.bfloat16)

        scores = jnp.dot(x_ref[:], rw_ref[:],
                         preferred_element_type=jnp.float32)
        s_max = jnp.max(scores, axis=1, keepdims=True)
        p = jnp.exp(scores - s_max)
        probs = p / jnp.sum(p, axis=1, keepdims=True)
        route = idx_ref[:]
        iota_e = lax.broadcasted_iota(jnp.int32, scores.shape, 1)
        prob_sel = jnp.sum(jnp.where(route == iota_e, probs, 0.0),
                           axis=1, keepdims=True)

        coeff_ref[:] = jnp.concatenate(
            [jnp.where(route == my * n_local_e + le, prob_sel, 0.0)
             for le in range(n_local_e)], axis=1)

        for j in range(N_DEV):
            dst = (my + 1 + j) % N_DEV
            xblk = x_ref[pl.ds(dst * m_per, m_per), :]
            cblk = coeff_ref[pl.ds(dst * m_per, m_per), :]
            xw_ref[j * m_per:(j + 1) * m_per, :] = jnp.concatenate(
                [xblk * cblk[:, le:le + 1] for le in range(n_local_e)],
                axis=1).astype(jnp.bfloat16)

        descs = []
        for c in (1, 2, 0, 3):
            r0 = c * rows_per_chunk
            pblk = jnp.zeros((rows_per_chunk, d_hidden), jnp.float32)
            for le in range(n_local_e):
                pblk = pblk + jnp.dot(
                    xw_ref[r0:r0 + rows_per_chunk,
                           le * d_model:(le + 1) * d_model],
                    ew_bf_ref[le],
                    preferred_element_type=jnp.float32)
            sendbuf_ref[r0:r0 + rows_per_chunk, :] = pblk.astype(jnp.bfloat16)
            for j in range(c * blocks_per_chunk, (c + 1) * blocks_per_chunk):
                k = j + 1
                if k < N_DEV:
                    pl.semaphore_wait(ready_sems.at[k], 1)
                    desc = pltpu.make_async_remote_copy(
                        src_ref=sendbuf_ref.at[pl.ds(j * m_per, m_per)],
                        dst_ref=gather_ref.at[k],
                        send_sem=send_sems.at[k],
                        recv_sem=recv_sems.at[k],
                        device_id=((my + k) % N_DEV,),
                        device_id_type=pl.DeviceIdType.MESH,
                    )
                    desc.start()
                    descs.append(desc)
                else:
                    gather_ref[0] = sendbuf_ref[j * m_per:(j + 1) * m_per, :]

        shared_blk = jnp.dot(x_ref[pl.ds(my * m_per, m_per), :], sw_ref[:],
                             preferred_element_type=jnp.float32)

        for desc in descs:
            desc.wait_recv()
        for desc in descs:
            desc.wait_send()

        out_ref[:] = shared_blk + jnp.sum(
            gather_ref[:].astype(jnp.float32), axis=0)

    return pl.pallas_call(
        body,
        out_shape=jax.ShapeDtypeStruct((m_per, d_hidden), jnp.float32),
        in_specs=[pl.BlockSpec(memory_space=pltpu.VMEM)] * 5,
        out_specs=pl.BlockSpec(memory_space=pltpu.VMEM),
        scratch_shapes=[
            pltpu.VMEM((n_tokens, n_local_e), jnp.float32),
            pltpu.VMEM((n_tokens, n_local_e * d_model), jnp.bfloat16),
            pltpu.VMEM((n_local_e, d_model, d_hidden), jnp.bfloat16),
            pltpu.VMEM((n_tokens, d_hidden), jnp.bfloat16),
            pltpu.VMEM((N_DEV, m_per, d_hidden), jnp.bfloat16),
            pltpu.SemaphoreType.DMA((N_DEV,)),
            pltpu.SemaphoreType.DMA((N_DEV,)),
            pltpu.SemaphoreType.REGULAR((N_DEV,)),
        ],
        compiler_params=pltpu.CompilerParams(collective_id=0),
    )(x, router_W, route_idx, expert_W, shared_W)
